# baseline (speedup 1.0000x reference)
import functools

import jax
import jax.numpy as jnp
from jax.experimental import pallas as pl
from jax.experimental.pallas import tpu as pltpu

_DROP_PROB = 0.0
_NUM_GRAPHS = 256
_ROWS = 5000
_ROWS_PAD = 5120


def _body(batch_ref, mask_ref, x_ref, o_ref):
    tbl_lo = jnp.broadcast_to(mask_ref[0:1, 0:128], (8, 128))
    tbl_hi = jnp.broadcast_to(mask_ref[0:1, 128:256], (8, 128))
    ids = batch_ref[0]
    pieces = []
    for c in range(_ROWS_PAD // 128):
        id8 = jnp.broadcast_to(ids[c:c + 1, :], (8, 128))
        idx7 = jnp.bitwise_and(id8, 127)
        s_lo = jnp.take_along_axis(tbl_lo, idx7, axis=1)
        s_hi = jnp.take_along_axis(tbl_hi, idx7, axis=1)
        pieces.append(jnp.where(id8 < 128, s_lo, s_hi)[0:1, :])
    scale_lane = jnp.concatenate(pieces, axis=1)[:, :_ROWS]
    scale = jax.lax.dot_general(scale_lane, jnp.ones((1, 1), jnp.float32),
                                (((0,), (0,)), ((), ())),
                                preferred_element_type=jnp.float32)
    o_ref[...] = x_ref[...] * scale


@functools.partial(jax.jit, static_argnames=())
def kernel(x, batch):
    n, d = x.shape
    num_blocks = n // _ROWS
    keep_prob = 1.0 - _DROP_PROB
    keep_mask = jnp.full((1, _NUM_GRAPHS), 1.0 / keep_prob, dtype=x.dtype)

    batch_pad = jnp.pad(batch.reshape(num_blocks, 1, _ROWS),
                        ((0, 0), (0, 0), (0, _ROWS_PAD - _ROWS)))
    batch_pad = batch_pad.reshape(num_blocks, _ROWS_PAD // 128, 128)

    return pl.pallas_call(
        _body,
        grid=(num_blocks,),
        in_specs=[
            pl.BlockSpec((1, _ROWS_PAD // 128, 128), lambda i: (i, 0, 0)),
            pl.BlockSpec((1, _NUM_GRAPHS), lambda i: (0, 0)),
            pl.BlockSpec((_ROWS, d), lambda i: (i, 0)),
        ],
        out_specs=pl.BlockSpec((_ROWS, d), lambda i: (i, 0)),
        out_shape=jax.ShapeDtypeStruct((n, d), x.dtype),
        compiler_params=pltpu.CompilerParams(
            fuse_transposed_lhs_in_matmul=True,
        ),
    )(batch_pad, keep_mask, x)


# speedup vs baseline: 1.0185x; 1.0185x over previous
import functools

import jax
import jax.numpy as jnp
from jax.experimental import pallas as pl
from jax.experimental.pallas import tpu as pltpu

_DROP_PROB = 0.0
_NUM_GRAPHS = 256
_ROWS = 5000
_ROWS_PAD = 5120


def _body(batch_ref, mask_ref, x_ref, o_ref):
    tbl_lo = jnp.broadcast_to(mask_ref[0:1, 0:128], (8, 128))
    tbl_hi = jnp.broadcast_to(mask_ref[0:1, 128:256], (8, 128))
    base = pl.program_id(0) * (_ROWS_PAD // 128)
    pieces = []
    for c in range(_ROWS_PAD // 128):
        id8 = jnp.broadcast_to(batch_ref[pl.ds(base + c, 1), :], (8, 128))
        idx7 = jnp.bitwise_and(id8, 127)
        s_lo = jnp.take_along_axis(tbl_lo, idx7, axis=1)
        s_hi = jnp.take_along_axis(tbl_hi, idx7, axis=1)
        pieces.append(jnp.where(id8 < 128, s_lo, s_hi)[0:1, :])
    scale_lane = jnp.concatenate(pieces, axis=1)[:, :_ROWS]
    scale = jax.lax.dot_general(scale_lane, jnp.ones((1, 1), jnp.float32),
                                (((0,), (0,)), ((), ())),
                                preferred_element_type=jnp.float32)
    o_ref[...] = x_ref[...] * scale


@functools.partial(jax.jit, static_argnames=())
def kernel(x, batch):
    n, d = x.shape
    num_blocks = n // _ROWS
    keep_prob = 1.0 - _DROP_PROB
    keep_mask = jnp.full((1, _NUM_GRAPHS), 1.0 / keep_prob, dtype=x.dtype)

    batch_pad = jnp.pad(batch.reshape(num_blocks, 1, _ROWS),
                        ((0, 0), (0, 0), (0, _ROWS_PAD - _ROWS)))
    batch_pad = batch_pad.reshape(num_blocks * (_ROWS_PAD // 128), 128)

    return pl.pallas_call(
        _body,
        grid=(num_blocks,),
        in_specs=[
            pl.BlockSpec((num_blocks * (_ROWS_PAD // 128), 128),
                         lambda i: (0, 0)),
            pl.BlockSpec((1, _NUM_GRAPHS), lambda i: (0, 0)),
            pl.BlockSpec((_ROWS, d), lambda i: (i, 0)),
        ],
        out_specs=pl.BlockSpec((_ROWS, d), lambda i: (i, 0)),
        out_shape=jax.ShapeDtypeStruct((n, d), x.dtype),
        compiler_params=pltpu.CompilerParams(
            fuse_transposed_lhs_in_matmul=True,
        ),
    )(batch_pad, keep_mask, x)
